# baseline (speedup 1.0000x reference)
import functools

import jax
import jax.numpy as jnp
from jax import lax
from jax.experimental import pallas as pl
from jax.experimental.pallas import tpu as pltpu
from jax.experimental.pallas import tpu_sc as plsc

D = 1024
V = 256
NC, NS = 2, 16
NW = NC * NS
CHUNK = 16
NBUF = 4
N_SC = 8192
BM = 2048


def _table_body(base_ref, proj_ref, w_ref, whi_ref, wlo_ref):
    r = lax.broadcasted_iota(jnp.int32, (V, 8), 0)
    j = lax.broadcasted_iota(jnp.int32, (V, 8), 1)
    bits = ((r >> (7 - j)) & 1).astype(jnp.float32)
    w = base_ref[...] + lax.dot_general(
        bits, proj_ref[...], (((1,), (1,)), ((), ())),
        preferred_element_type=jnp.float32)
    w_ref[...] = w
    hi = w.astype(jnp.bfloat16)
    whi_ref[...] = hi
    wlo_ref[...] = (w - hi.astype(jnp.float32)).astype(jnp.bfloat16)


def _build_table(base_weight, bit_proj_w):
    return pl.pallas_call(
        _table_body,
        out_shape=(
            jax.ShapeDtypeStruct((V, D), jnp.float32),
            jax.ShapeDtypeStruct((V, D), jnp.bfloat16),
            jax.ShapeDtypeStruct((V, D), jnp.bfloat16),
        ),
    )(base_weight, bit_proj_w)


def _make_sc_gather(total_ids):
    b_per_w = N_SC // NW
    n_chunks = b_per_w // CHUNK
    assert n_chunks % NBUF == 0 and n_chunks >= 2 * NBUF
    mesh = plsc.VectorSubcoreMesh(
        core_axis_name="c", subcore_axis_name="s",
        num_cores=NC, num_subcores=NS)

    @functools.partial(
        pl.kernel,
        mesh=mesh,
        out_type=jax.ShapeDtypeStruct((total_ids, D), jnp.float32),
        scratch_types=[
            pltpu.VMEM((b_per_w,), jnp.int32),
            pltpu.VMEM((CHUNK, D), jnp.float32),
            pltpu.VMEM((CHUNK, D), jnp.float32),
            pltpu.VMEM((CHUNK, D), jnp.float32),
            pltpu.VMEM((CHUNK, D), jnp.float32),
            pltpu.SemaphoreType.DMA,
            pltpu.SemaphoreType.DMA,
        ],
    )
    def gather_k(table_hbm, ids_hbm, out_hbm, idx_v, b0, b1, b2, b3,
                 gsem, ssem):
        bufs = (b0, b1, b2, b3)
        wid = lax.axis_index("s") * NC + lax.axis_index("c")
        base = wid * b_per_w
        pltpu.sync_copy(ids_hbm.at[pl.ds(base, b_per_w)], idx_v)

        def start_g(c, buf):
            pltpu.async_copy(
                table_hbm.at[idx_v.at[pl.ds(c * CHUNK, CHUNK)]], buf, gsem)

        def start_s(c, buf):
            pltpu.async_copy(
                buf, out_hbm.at[pl.ds(base + c * CHUNK, CHUNK)], ssem)

        def wait_g():
            pltpu.make_async_copy(
                table_hbm.at[pl.ds(0, CHUNK)], b0, gsem).wait()

        def wait_s():
            pltpu.make_async_copy(
                b0, out_hbm.at[pl.ds(base, CHUNK)], ssem).wait()

        start_g(0, b0)
        start_g(1, b1)
        wait_g()
        start_s(0, b0)
        start_g(2, b2)
        wait_g()
        start_s(1, b1)
        start_g(3, b3)

        def body(j, carry):
            for b in range(NBUF):
                c = 2 + NBUF * j + b
                wait_g()
                start_s(c, bufs[(b + 2) % NBUF])
                wait_s()
                start_g(c + 2, bufs[b])
            return carry

        lax.fori_loop(0, (n_chunks - 4) // NBUF, body, 0, unroll=False)

        wait_g()
        start_s(n_chunks - 2, bufs[(n_chunks - 2) % NBUF])
        wait_s()
        wait_g()
        start_s(n_chunks - 1, bufs[(n_chunks - 1) % NBUF])
        wait_s()
        wait_s()
        wait_s()

    return gather_k


def _onehot_body(ids_ref, whi_ref, wlo_ref, alias_ref, out_ref):
    del alias_ref
    idv = ids_ref[0, 0, :]
    col = lax.broadcasted_iota(jnp.int32, (BM, V), 1)
    oh = (idv[:, None] == col).astype(jnp.bfloat16)
    acc = jnp.dot(oh, whi_ref[...], preferred_element_type=jnp.float32)
    acc = acc + jnp.dot(oh, wlo_ref[...], preferred_element_type=jnp.float32)
    out_ref[...] = acc


def _tc_fill(ids_tc, whi, wlo, partial_out):
    total, _ = partial_out.shape
    n_tc = ids_tc.shape[0]
    assert n_tc % BM == 0 and N_SC % BM == 0
    grid = n_tc // BM
    ids3 = ids_tc.reshape(grid, 1, BM)
    return pl.pallas_call(
        _onehot_body,
        grid=(grid,),
        in_specs=[
            pl.BlockSpec((1, 1, BM), lambda i: (i, 0, 0)),
            pl.BlockSpec((V, D), lambda i: (0, 0)),
            pl.BlockSpec((V, D), lambda i: (0, 0)),
            pl.BlockSpec(memory_space=pl.ANY),
        ],
        out_specs=pl.BlockSpec((BM, D), lambda i: (i + N_SC // BM, 0)),
        out_shape=jax.ShapeDtypeStruct((total, D), jnp.float32),
        input_output_aliases={3: 0},
    )(ids3, whi, wlo, partial_out)


def kernel(input_ids, base_weight, bit_proj_w):
    bsz, seq = input_ids.shape
    total = bsz * seq
    w, whi, wlo = _build_table(base_weight, bit_proj_w)
    ids = input_ids.reshape(-1).astype(jnp.int32)
    sc_out = _make_sc_gather(total)(w, ids)
    out = _tc_fill(ids[N_SC:], whi, wlo, sc_out)
    return out.reshape(bsz, seq, D)


# speedup vs baseline: 1.1114x; 1.1114x over previous
import functools

import jax
import jax.numpy as jnp
from jax import lax
from jax.experimental import pallas as pl
from jax.experimental.pallas import tpu as pltpu
from jax.experimental.pallas import tpu_sc as plsc

D = 1024
V = 256
NC, NS = 2, 16
NW = NC * NS
CHUNK = 16
NBUF = 4
N_SC = 4096
BM = 2048


def _table_body(base_ref, proj_ref, w_ref, whi_ref, wlo_ref):
    r = lax.broadcasted_iota(jnp.int32, (V, 8), 0)
    j = lax.broadcasted_iota(jnp.int32, (V, 8), 1)
    bits = ((r >> (7 - j)) & 1).astype(jnp.float32)
    w = base_ref[...] + lax.dot_general(
        bits, proj_ref[...], (((1,), (1,)), ((), ())),
        preferred_element_type=jnp.float32)
    w_ref[...] = w
    hi = w.astype(jnp.bfloat16)
    whi_ref[...] = hi
    wlo_ref[...] = (w - hi.astype(jnp.float32)).astype(jnp.bfloat16)


def _build_table(base_weight, bit_proj_w):
    return pl.pallas_call(
        _table_body,
        out_shape=(
            jax.ShapeDtypeStruct((V, D), jnp.float32),
            jax.ShapeDtypeStruct((V, D), jnp.bfloat16),
            jax.ShapeDtypeStruct((V, D), jnp.bfloat16),
        ),
    )(base_weight, bit_proj_w)


def _make_sc_gather(total_ids):
    b_per_w = N_SC // NW
    n_chunks = b_per_w // CHUNK
    assert n_chunks % NBUF == 0 and n_chunks >= 2 * NBUF
    mesh = plsc.VectorSubcoreMesh(
        core_axis_name="c", subcore_axis_name="s",
        num_cores=NC, num_subcores=NS)

    @functools.partial(
        pl.kernel,
        mesh=mesh,
        out_type=jax.ShapeDtypeStruct((total_ids, D), jnp.float32),
        scratch_types=[
            pltpu.VMEM((b_per_w,), jnp.int32),
            pltpu.VMEM((CHUNK, D), jnp.float32),
            pltpu.VMEM((CHUNK, D), jnp.float32),
            pltpu.VMEM((CHUNK, D), jnp.float32),
            pltpu.VMEM((CHUNK, D), jnp.float32),
            pltpu.SemaphoreType.DMA,
            pltpu.SemaphoreType.DMA,
        ],
    )
    def gather_k(table_hbm, ids_hbm, out_hbm, idx_v, b0, b1, b2, b3,
                 gsem, ssem):
        bufs = (b0, b1, b2, b3)
        wid = lax.axis_index("s") * NC + lax.axis_index("c")
        base = wid * b_per_w
        pltpu.sync_copy(ids_hbm.at[pl.ds(base, b_per_w)], idx_v)

        def start_g(c, buf):
            pltpu.async_copy(
                table_hbm.at[idx_v.at[pl.ds(c * CHUNK, CHUNK)]], buf, gsem)

        def start_s(c, buf):
            pltpu.async_copy(
                buf, out_hbm.at[pl.ds(base + c * CHUNK, CHUNK)], ssem)

        def wait_g():
            pltpu.make_async_copy(
                table_hbm.at[pl.ds(0, CHUNK)], b0, gsem).wait()

        def wait_s():
            pltpu.make_async_copy(
                b0, out_hbm.at[pl.ds(base, CHUNK)], ssem).wait()

        start_g(0, b0)
        start_g(1, b1)
        wait_g()
        start_s(0, b0)
        start_g(2, b2)
        wait_g()
        start_s(1, b1)
        start_g(3, b3)

        def body(j, carry):
            for b in range(NBUF):
                c = 2 + NBUF * j + b
                wait_g()
                start_s(c, bufs[(b + 2) % NBUF])
                wait_s()
                start_g(c + 2, bufs[b])
            return carry

        lax.fori_loop(0, (n_chunks - 4) // NBUF, body, 0, unroll=False)

        wait_g()
        start_s(n_chunks - 2, bufs[(n_chunks - 2) % NBUF])
        wait_s()
        wait_g()
        start_s(n_chunks - 1, bufs[(n_chunks - 1) % NBUF])
        wait_s()
        wait_s()
        wait_s()

    return gather_k


def _onehot_body(ids_ref, whi_ref, wlo_ref, alias_ref, out_ref):
    del alias_ref
    idv = ids_ref[0, 0, :]
    col = lax.broadcasted_iota(jnp.int32, (BM, V), 1)
    oh = (idv[:, None] == col).astype(jnp.bfloat16)
    acc = jnp.dot(oh, whi_ref[...], preferred_element_type=jnp.float32)
    acc = acc + jnp.dot(oh, wlo_ref[...], preferred_element_type=jnp.float32)
    out_ref[...] = acc


def _tc_fill(ids_tc, whi, wlo, partial_out):
    total, _ = partial_out.shape
    n_tc = ids_tc.shape[0]
    assert n_tc % BM == 0 and N_SC % BM == 0
    grid = n_tc // BM
    ids3 = ids_tc.reshape(grid, 1, BM)
    return pl.pallas_call(
        _onehot_body,
        grid=(grid,),
        in_specs=[
            pl.BlockSpec((1, 1, BM), lambda i: (i, 0, 0)),
            pl.BlockSpec((V, D), lambda i: (0, 0)),
            pl.BlockSpec((V, D), lambda i: (0, 0)),
            pl.BlockSpec(memory_space=pl.ANY),
        ],
        out_specs=pl.BlockSpec((BM, D), lambda i: (i + N_SC // BM, 0)),
        out_shape=jax.ShapeDtypeStruct((total, D), jnp.float32),
        input_output_aliases={3: 0},
    )(ids3, whi, wlo, partial_out)


def kernel(input_ids, base_weight, bit_proj_w):
    bsz, seq = input_ids.shape
    total = bsz * seq
    w, whi, wlo = _build_table(base_weight, bit_proj_w)
    ids = input_ids.reshape(-1).astype(jnp.int32)
    sc_out = _make_sc_gather(total)(w, ids)
    out = _tc_fill(ids[N_SC:], whi, wlo, sc_out)
    return out.reshape(bsz, seq, D)
